# Initial kernel scaffold; baseline (speedup 1.0000x reference)
import jax
import jax.numpy as jnp
from jax.experimental import pallas as pl


def kernel(x, router_w, w1, w2):
    raise NotImplementedError("write your pallas kernel here")


# speedup vs baseline: 3.8381x; 3.8381x over previous
import functools

import numpy as np
import jax
import jax.numpy as jnp
from jax import lax
from jax.experimental import pallas as pl
from jax.experimental.pallas import tpu as pltpu
from jax.experimental.pallas import tpu_sc as plsc

_B, _T, _C = 4, 8192, 768
_N = _B * _T
_E = 64
_D = 128
_K = _N // 64
_TN = 2048
_LANES = 16
_NW = 32
_ROWS_PER_W = (_E // _NW) * _K
_GCH = 64


def _monotone_key(logits_f32):
    b = lax.bitcast_convert_type(logits_f32, jnp.int32)
    return b ^ ((b >> 31) & jnp.int32(0x7FFFFFFF))


def _k1_body(x_ref, rw_ref, logt_ref, thr_ref, cnt_ref, keys_scr):
    step = pl.program_id(0)
    x_blk = x_ref[...]
    rw = rw_ref[...]
    logits = lax.dot_general(x_blk, rw, (((1,), (1,)), ((), ())))
    keys = _monotone_key(logits)
    keys_scr[pl.ds(step * _TN, _TN), :] = keys
    eye = (lax.broadcasted_iota(jnp.int32, (_E, _E), 0)
           == lax.broadcasted_iota(jnp.int32, (_E, _E), 1)).astype(jnp.float32)
    logt_ref[...] = lax.dot_general(
        eye, logits, (((1,), (1,)), ((), ())),
        precision=lax.Precision.HIGHEST)

    @pl.when(step == pl.num_programs(0) - 1)
    def _search():
        n_sub, sub = 8, _N // 8

        def count_ge(cand, strict):
            def chunk(ci, acc):
                blk = keys_scr[pl.ds(ci * sub, sub), :]
                m = (blk > cand) if strict else (blk >= cand)
                return acc + jnp.sum(m.astype(jnp.int32), axis=0,
                                     keepdims=True)
            return lax.fori_loop(0, n_sub, chunk,
                                 jnp.zeros((1, _E), jnp.int32))

        def bit_step(b, s):
            bit = jnp.int32(31) - b
            cand = s ^ lax.shift_left(jnp.int32(1), bit)
            return jnp.where(count_ge(cand, False) >= _K, cand, s)

        s0 = jnp.full((1, _E), jnp.iinfo(jnp.int32).min, jnp.int32)
        s_fin = lax.fori_loop(0, 32, bit_step, s0)
        cnt_gt = count_ge(s_fin, True)
        thr_ref[...] = jnp.broadcast_to(s_fin, (8, _E))
        cnt_ref[...] = jnp.broadcast_to(cnt_gt, (8, _E))


def _run_k1(x_flat, router_w):
    return pl.pallas_call(
        _k1_body,
        grid=(_N // _TN,),
        in_specs=[
            pl.BlockSpec((_TN, _C), lambda i: (i, 0)),
            pl.BlockSpec((_E, _C), lambda i: (0, 0)),
        ],
        out_specs=[
            pl.BlockSpec((_E, _TN), lambda i: (0, i)),
            pl.BlockSpec((8, _E), lambda i: (0, 0)),
            pl.BlockSpec((8, _E), lambda i: (0, 0)),
        ],
        out_shape=[
            jax.ShapeDtypeStruct((_E, _N), jnp.float32),
            jax.ShapeDtypeStruct((8, _E), jnp.int32),
            jax.ShapeDtypeStruct((8, _E), jnp.int32),
        ],
        scratch_shapes=[pltpu.VMEM((_N, _E), jnp.int32)],
    )(x_flat, router_w)


def _extract_bcast(vec16, pos):
    lane = lax.broadcasted_iota(jnp.int32, (_LANES,), 0)
    return jnp.max(jnp.where(lane == pos, vec16, jnp.iinfo(jnp.int32).min))


def _compact_one_expert(lrow_ref, cidx_ref, cval_ref, slot, s_thr, cnt_gt):
    base_out = slot * _K
    s_vec = jnp.full((_LANES,), s_thr, jnp.int32)
    lane = lax.broadcasted_iota(jnp.int32, (_LANES,), 0)
    ones_lim = base_out + _K

    def body(i, carry):
        gt_ptr, eq_ptr = carry
        v = lrow_ref[pl.ds(i * _LANES, _LANES)]
        kb = _monotone_key(v)
        m_ge = kb >= s_vec
        n_ge = jnp.sum(m_ge.astype(jnp.int32))

        def hit(ptrs):
            g, e = ptrs
            ids = lane + i * _LANES
            m_gt = kb > s_vec
            pre_gt = plsc.cumsum(m_gt.astype(jnp.int32))
            pos_gt = g + pre_gt - 1 + base_out
            plsc.store_scatter(cidx_ref, [pos_gt], ids, mask=m_gt)
            plsc.store_scatter(cval_ref, [pos_gt], v, mask=m_gt)
            m_eq = kb == s_vec
            pre_eq = plsc.cumsum(m_eq.astype(jnp.int32))
            pos_eq = e + pre_eq - 1 + base_out
            m_acc = m_eq & (pos_eq < ones_lim)
            plsc.store_scatter(cidx_ref, [pos_eq], ids, mask=m_acc)
            plsc.store_scatter(cval_ref, [pos_eq], v, mask=m_acc)
            n_gt = jnp.sum(m_gt.astype(jnp.int32))
            return (g + n_gt, e + (n_ge - n_gt))

        return lax.cond(n_ge > 0, hit, lambda p: p, (gt_ptr, eq_ptr))

    lax.fori_loop(0, _N // _LANES, body, (jnp.int32(0), cnt_gt))


def _k2_body(logt_hbm, thr_hbm, cnt_hbm, cval_hbm, cidx_hbm, fan_hbm,
             l0_v, l1_v, cidx_v, cval_v, fan_v, thr_v, cnt_v, sem0, sem1):
    cid = lax.axis_index("c")
    sid = lax.axis_index("s")
    wid = sid * 2 + cid
    e0 = wid * 2

    cp0 = pltpu.async_copy(logt_hbm.at[e0], l0_v, sem0)
    cp1 = pltpu.async_copy(logt_hbm.at[e0 + 1], l1_v, sem1)
    pltpu.sync_copy(thr_hbm.at[0], thr_v)
    pltpu.sync_copy(cnt_hbm.at[0], cnt_v)

    grp = (e0 // _LANES) * _LANES
    thr16 = thr_v[pl.ds(grp, _LANES)]
    cnt16 = cnt_v[pl.ds(grp, _LANES)]
    p0 = e0 - grp
    s0 = _extract_bcast(thr16, p0)
    s1 = _extract_bcast(thr16, p0 + 1)
    c0 = _extract_bcast(cnt16, p0)
    c1 = _extract_bcast(cnt16, p0 + 1)

    zeros = jnp.zeros((_LANES,), jnp.float32)

    def zbody(i, _):
        fan_v[pl.ds(i * _LANES, _LANES)] = zeros
        return 0

    lax.fori_loop(0, _N // _LANES, zbody, 0)

    cp0.wait()
    _compact_one_expert(l0_v, cidx_v, cval_v, 0, s0, c0)
    cp1.wait()
    _compact_one_expert(l1_v, cidx_v, cval_v, 1, s1, c1)

    ones = jnp.ones((_LANES,), jnp.float32)

    def fbody(j, _):
        ids = cidx_v[pl.ds(j * _LANES, _LANES)]
        plsc.addupdate_scatter(fan_v, [ids], ones)
        return 0

    lax.fori_loop(0, (2 * _K) // _LANES, fbody, 0)

    pltpu.sync_copy(cval_v.at[pl.ds(0, _K)], cval_hbm.at[e0])
    pltpu.sync_copy(cval_v.at[pl.ds(_K, _K)], cval_hbm.at[e0 + 1])
    pltpu.sync_copy(cidx_v.at[pl.ds(0, _K)], cidx_hbm.at[e0])
    pltpu.sync_copy(cidx_v.at[pl.ds(_K, _K)], cidx_hbm.at[e0 + 1])
    pltpu.sync_copy(fan_v, fan_hbm.at[wid])


def _run_k2(logits_t, thr, cnt):
    mesh = plsc.VectorSubcoreMesh(core_axis_name="c", subcore_axis_name="s",
                                  num_cores=2, num_subcores=16)
    kfn = pl.kernel(
        _k2_body,
        out_type=[
            jax.ShapeDtypeStruct((_E, _K), jnp.float32),
            jax.ShapeDtypeStruct((_E, _K), jnp.int32),
            jax.ShapeDtypeStruct((_NW, _N), jnp.float32),
        ],
        mesh=mesh,
        scratch_types=[
            pltpu.VMEM((_N,), jnp.float32),
            pltpu.VMEM((_N,), jnp.float32),
            pltpu.VMEM((2 * _K,), jnp.int32),
            pltpu.VMEM((2 * _K,), jnp.float32),
            pltpu.VMEM((_N,), jnp.float32),
            pltpu.VMEM((_E,), jnp.int32),
            pltpu.VMEM((_E,), jnp.int32),
            pltpu.SemaphoreType.DMA,
            pltpu.SemaphoreType.DMA,
        ],
        compiler_params=pltpu.CompilerParams(needs_layout_passes=False),
    )
    return kfn(logits_t, thr, cnt)


def _k3_body(cval_ref, cidx_ref, fan_ref, idx_ref, wgt_ref, fanout_ref):
    v = cval_ref[...]
    ix = cidx_ref[...]
    pos = lax.broadcasted_iota(jnp.int32, (_E, _K), 1)

    for ksz_exp in range(1, 10):
        ksz = 1 << ksz_exp
        dir_desc = (pos & ksz) == 0
        if ksz == _K:
            dir_desc = jnp.full((_E, _K), True)
        for j_exp in range(ksz_exp - 1, -1, -1):
            j = 1 << j_exp
            is_lo = (pos & j) == 0
            pv = jnp.where(is_lo, jnp.roll(v, -j, axis=1),
                           jnp.roll(v, j, axis=1))
            pi = jnp.where(is_lo, jnp.roll(ix, -j, axis=1),
                           jnp.roll(ix, j, axis=1))
            lo_v = jnp.where(is_lo, v, pv)
            hi_v = jnp.where(is_lo, pv, v)
            lo_i = jnp.where(is_lo, ix, pi)
            hi_i = jnp.where(is_lo, pi, ix)
            good = (lo_v > hi_v) | ((lo_v == hi_v) & (lo_i < hi_i))
            swap = good ^ dir_desc
            v = jnp.where(swap, pv, v)
            ix = jnp.where(swap, pi, ix)

    idx_ref[...] = ix
    wgt_ref[...] = 1.0 / (1.0 + jnp.exp(-v))
    fanout_ref[...] = jnp.sum(fan_ref[...], axis=0, keepdims=True)


def _run_k3(cval, cidx, fan_part):
    return pl.pallas_call(
        _k3_body,
        out_shape=[
            jax.ShapeDtypeStruct((_E, _K), jnp.int32),
            jax.ShapeDtypeStruct((_E, _K), jnp.float32),
            jax.ShapeDtypeStruct((1, _N), jnp.float32),
        ],
    )(cval, cidx, fan_part)


def _k4_body(x_hbm, idx_hbm, out_hbm, idx_v, rows_v, sems):
    cid = lax.axis_index("c")
    sid = lax.axis_index("s")
    wid = sid * 2 + cid
    e0 = wid * 2
    pltpu.sync_copy(idx_hbm.at[pl.ds(e0, 2)], idx_v)

    n_ch = _ROWS_PER_W // _GCH
    per_row = _K // _GCH

    def chunk_idx_ref(c):
        return idx_v.at[c // per_row, pl.ds((c % per_row) * _GCH, _GCH)]

    cps = [None, None]
    cps[0] = pltpu.async_copy(x_hbm.at[chunk_idx_ref(0)], rows_v.at[0],
                              sems.at[0])
    for c in range(n_ch):
        buf = c % 2
        nbuf = (c + 1) % 2
        cps[buf].wait()
        if c + 1 < n_ch:
            cps[nbuf] = pltpu.async_copy(
                x_hbm.at[chunk_idx_ref(c + 1)], rows_v.at[nbuf],
                sems.at[nbuf])
        base = wid * _ROWS_PER_W + c * _GCH
        pltpu.sync_copy(rows_v.at[buf], out_hbm.at[pl.ds(base, _GCH)])


def _run_k4(x_flat, topk_idx):
    mesh = plsc.VectorSubcoreMesh(core_axis_name="c", subcore_axis_name="s",
                                  num_cores=2, num_subcores=16)
    kfn = pl.kernel(
        _k4_body,
        out_type=jax.ShapeDtypeStruct((_E * _K, _C), jnp.float32),
        mesh=mesh,
        scratch_types=[
            pltpu.VMEM((2, _K), jnp.int32),
            pltpu.VMEM((2, _GCH, _C), jnp.float32),
            pltpu.SemaphoreType.DMA((2,)),
        ],
        compiler_params=pltpu.CompilerParams(needs_layout_passes=False),
    )
    return kfn(x_flat, topk_idx)


def _k5_body(xe_ref, w1_ref, w2_ref, out_ref):
    xe = xe_ref[0]
    w1 = w1_ref[0]
    w2 = w2_ref[0]
    h = jnp.maximum(
        lax.dot_general(xe, w1, (((1,), (1,)), ((), ()))), 0.0)
    out_ref[0] = lax.dot_general(h, w2, (((1,), (1,)), ((), ())))


def _run_k5(x_e, w1, w2):
    return pl.pallas_call(
        _k5_body,
        grid=(_E,),
        in_specs=[
            pl.BlockSpec((1, _K, _C), lambda e: (e, 0, 0)),
            pl.BlockSpec((1, _D, _C), lambda e: (e, 0, 0)),
            pl.BlockSpec((1, _C, _D), lambda e: (e, 0, 0)),
        ],
        out_specs=pl.BlockSpec((1, _K, _C), lambda e: (e, 0, 0)),
        out_shape=jax.ShapeDtypeStruct((_E, _K, _C), jnp.float32),
    )(x_e.reshape(_E, _K, _C), w1, w2)


def kernel(x, router_w, w1, w2):
    b, t, c = x.shape
    x_flat = x.reshape(-1, c)
    logits_t, thr, cnt = _run_k1(x_flat, router_w)
    cval, cidx, fan_part = _run_k2(logits_t, thr, cnt)
    topk_idx, weights, fanout = _run_k3(cval, cidx, fan_part)
    x_e = _run_k4(x_flat, topk_idx)
    h = _run_k5(x_e, w1, w2)
    return (h.reshape(_E * _K, c), topk_idx.reshape(-1),
            weights.reshape(-1), fanout.reshape(-1))
